# baseline (speedup 1.0000x reference)
import functools

import jax
import jax.numpy as jnp
from jax import lax
from jax.experimental import pallas as pl
from jax.experimental.pallas import tpu as pltpu
from jax.experimental.pallas import tpu_sc as plsc

B = 16384
F = 26
V = 100000
D = 16

NC = 2
NS = 16
L = 16
NW = NC * NS

R = F * V
RFULL = R - R % 128
GW = 1024
NGRP = RFULL // GW
TAIL = R - RFULL

BPW = B // NW
CH = 128
NCH = BPW // CH
CF = CH * F
GS = 128
NG = CF // GS
NV = CF // L


def _reformat_body(emb_t_hbm, tail_hbm, out_hbm, in0, in1, tp0, tp1, tbuf,
                   isem0, isem1, osem0, osem1):
    wid = lax.axis_index("s") * NC + lax.axis_index("c")
    ibufs = (in0, in1)
    obufs = (tp0, tp1)
    isems = (isem0, isem1)
    osems = (osem0, osem1)

    n_i = (NGRP - wid + NW - 1) // NW
    rows = lax.iota(jnp.int32, L)

    def g_of(i):
        return wid + i * NW

    def start_in(i, slot):
        pltpu.async_copy(
            emb_t_hbm.at[:, pl.ds(g_of(i) * GW, GW)], ibufs[slot], isems[slot])

    def wait_in(i, slot):
        pltpu.make_async_copy(
            emb_t_hbm.at[:, pl.ds(g_of(i) * GW, GW)], ibufs[slot],
            isems[slot]).wait()

    def start_out(i, slot):
        pltpu.async_copy(
            obufs[slot], out_hbm.at[pl.ds(g_of(i) * GW * D, GW * D)],
            osems[slot])

    def wait_out(i, slot):
        pltpu.make_async_copy(
            obufs[slot], out_hbm.at[pl.ds(g_of(i) * GW * D, GW * D)],
            osems[slot]).wait()

    def main():
        start_in(0, 0)

        def grp(i, _):
            slot = lax.rem(i, 2)

            @pl.when(i + 1 < n_i)
            def _():
                @pl.when(slot == 0)
                def _():
                    start_in(i + 1, 1)

                @pl.when(slot == 1)
                def _():
                    start_in(i + 1, 0)

            @pl.when(i >= 2)
            def _():
                @pl.when(slot == 0)
                def _():
                    wait_out(i - 2, 0)

                @pl.when(slot == 1)
                def _():
                    wait_out(i - 2, 1)

            def do_slot(s):
                wait_in(i, s)
                src = ibufs[s]
                dst = obufs[s]

                @plsc.parallel_loop(0, GW, unroll=8)
                def _(rm):
                    cols = jnp.zeros((L,), jnp.int32) + rm
                    vals = plsc.load_gather(src, [rows, cols])
                    dst[pl.ds(rm * D, D)] = vals

                start_out(i, s)

            @pl.when(slot == 0)
            def _():
                do_slot(0)

            @pl.when(slot == 1)
            def _():
                do_slot(1)

            return 0

        lax.fori_loop(0, n_i, grp, 0)

        def drain_at(i):
            @pl.when(lax.rem(i, 2) == 0)
            def _():
                wait_out(i, 0)

            @pl.when(lax.rem(i, 2) == 1)
            def _():
                wait_out(i, 1)

        drain_at(n_i - 2)
        drain_at(n_i - 1)

    main()

    @pl.when(wid == NW - 1)
    def _():
        pltpu.sync_copy(tail_hbm, tbuf)
        pltpu.sync_copy(tbuf, out_hbm.at[pl.ds(RFULL * D, TAIL * D)])


_reformat = functools.partial(
    pl.kernel,
    mesh=plsc.VectorSubcoreMesh(core_axis_name="c", subcore_axis_name="s"),
    out_type=jax.ShapeDtypeStruct((R * D,), jnp.float32),
    scratch_types=[
        pltpu.VMEM((L, GW), jnp.float32),
        pltpu.VMEM((L, GW), jnp.float32),
        pltpu.VMEM((GW * D,), jnp.float32),
        pltpu.VMEM((GW * D,), jnp.float32),
        pltpu.VMEM((TAIL * D,), jnp.float32),
        pltpu.SemaphoreType.DMA,
        pltpu.SemaphoreType.DMA,
        pltpu.SemaphoreType.DMA,
        pltpu.SemaphoreType.DMA,
    ],
    compiler_params=pltpu.CompilerParams(
        needs_layout_passes=False, use_tc_tiling_on_sc=True),
)(_reformat_body)


def _fm_body(idx_hbm, emb_hbm, lin_hbm, out_hbm,
             idx_v, rows_v, lin_v, acc_v, gsem, lsem):
    wid = lax.axis_index("s") * NC + lax.axis_index("c")
    base = wid * BPW

    def chunk(c, _):
        cbase = base + c * CH
        pltpu.sync_copy(idx_hbm.at[pl.ds(cbase * F, CF)], idx_v)

        @plsc.parallel_loop(0, NV, unroll=8)
        def offs(j):
            p = j * L + lax.iota(jnp.int32, L)
            f = lax.rem(p, F)
            idx_v[pl.ds(j * L, L)] = idx_v[pl.ds(j * L, L)] + f * V

        def fire(g, _):
            pltpu.async_copy(emb_hbm.at[idx_v.at[pl.ds(g * GS, GS)]],
                             rows_v.at[pl.ds(g * GS, GS)], gsem)
            pltpu.async_copy(lin_hbm.at[idx_v.at[pl.ds(g * GS, GS)]],
                             lin_v.at[pl.ds(g * GS, GS)], lsem)
            return 0

        lax.fori_loop(0, NG, fire, 0)

        zv = jnp.zeros((L,), jnp.float32)

        @plsc.parallel_loop(0, CH // L)
        def init(j):
            acc_v[pl.ds(j * L, L)] = zv

        def drain(g, _):
            pltpu.make_async_copy(emb_hbm.at[idx_v.at[pl.ds(g * GS, GS)]],
                                  rows_v.at[pl.ds(g * GS, GS)], gsem).wait()
            pltpu.make_async_copy(lin_hbm.at[idx_v.at[pl.ds(g * GS, GS)]],
                                  lin_v.at[pl.ds(g * GS, GS)], lsem).wait()
            return 0

        lax.fori_loop(0, NG, drain, 0)

        def lin_pass(j, _):
            p = j * L + lax.iota(jnp.int32, L)
            sid = lax.div(p, F)
            plsc.addupdate_scatter(acc_v, [sid], lin_v[pl.ds(j * L, L)])
            return 0

        lax.fori_loop(0, NV, lin_pass, 0)

        @plsc.parallel_loop(0, CH, unroll=2)
        def sample(b):
            s = jnp.zeros((L,), jnp.float32)
            q = jnp.zeros((L,), jnp.float32)
            for f in range(F):
                r = rows_v[b * F + f, :]
                s = s + r
                q = q + r * r
            t = (s * s - q) * 0.5
            sid = jnp.zeros((L,), jnp.int32) + b
            plsc.addupdate_scatter(acc_v, [sid], t)

        pltpu.sync_copy(acc_v, out_hbm.at[pl.ds(cbase, CH)])
        return 0

    lax.fori_loop(0, NCH, chunk, 0)


_fm = functools.partial(
    pl.kernel,
    mesh=plsc.VectorSubcoreMesh(core_axis_name="c", subcore_axis_name="s"),
    out_type=jax.ShapeDtypeStruct((B,), jnp.float32),
    scratch_types=[
        pltpu.VMEM((CF,), jnp.int32),
        pltpu.VMEM((CF, D), jnp.float32),
        pltpu.VMEM((CF,), jnp.float32),
        pltpu.VMEM((CH,), jnp.float32),
        pltpu.SemaphoreType.DMA,
        pltpu.SemaphoreType.DMA,
    ],
    compiler_params=pltpu.CompilerParams(
        needs_layout_passes=False, use_tc_tiling_on_sc=False),
)(_fm_body)


@jax.jit
def kernel(inputs, emb_table, lin_table, bias):
    idx_flat = inputs.astype(jnp.int32).reshape(B * F)
    lin_flat = lin_table.reshape(F * V)
    tail_rows = emb_table[RFULL:, :].reshape(TAIL * D)
    table_flat = _reformat(emb_table.T, tail_rows)
    table = table_flat.reshape(F * V, D)
    out = _fm(idx_flat, table, lin_flat)
    return out.reshape(B, 1) + bias


# speedup vs baseline: 1.0996x; 1.0996x over previous
import functools

import jax
import jax.numpy as jnp
from jax import lax
from jax.experimental import pallas as pl
from jax.experimental.pallas import tpu as pltpu
from jax.experimental.pallas import tpu_sc as plsc

B = 16384
F = 26
V = 100000
D = 16

NC = 2
NS = 16
L = 16
NW = NC * NS

R = F * V
RFULL = R - R % 128
GW = 1024
NGRP = RFULL // GW
TAIL = R - RFULL

BPW = B // NW
CH = 128
NCH = BPW // CH
CF = CH * F
GS = 128
NG = CF // GS
NV = CF // L


def _reformat_body(emb_t_hbm, tail_hbm, out_hbm, in0, in1, tp0, tp1, tbuf,
                   isem0, isem1, osem0, osem1):
    wid = lax.axis_index("s") * NC + lax.axis_index("c")
    ibufs = (in0, in1)
    obufs = (tp0, tp1)
    isems = (isem0, isem1)
    osems = (osem0, osem1)

    n_i = (NGRP - wid + NW - 1) // NW
    rows = lax.iota(jnp.int32, L)

    def g_of(i):
        return wid + i * NW

    def start_in(i, slot):
        for c in range(D):
            pltpu.async_copy(
                emb_t_hbm.at[c, pl.ds(g_of(i) * GW, GW)],
                ibufs[slot].at[pl.ds(c * GW, GW)], isems[slot])

    def wait_in(i, slot):
        for c in range(D):
            pltpu.make_async_copy(
                emb_t_hbm.at[c, pl.ds(g_of(i) * GW, GW)],
                ibufs[slot].at[pl.ds(c * GW, GW)], isems[slot]).wait()

    def start_out(i, slot):
        pltpu.async_copy(
            obufs[slot], out_hbm.at[pl.ds(g_of(i) * GW * D, GW * D)],
            osems[slot])

    def wait_out(i, slot):
        pltpu.make_async_copy(
            obufs[slot], out_hbm.at[pl.ds(g_of(i) * GW * D, GW * D)],
            osems[slot]).wait()

    def main():
        start_in(0, 0)

        def grp(i, _):
            slot = lax.rem(i, 2)

            @pl.when(i + 1 < n_i)
            def _():
                @pl.when(slot == 0)
                def _():
                    start_in(i + 1, 1)

                @pl.when(slot == 1)
                def _():
                    start_in(i + 1, 0)

            @pl.when(i >= 2)
            def _():
                @pl.when(slot == 0)
                def _():
                    wait_out(i - 2, 0)

                @pl.when(slot == 1)
                def _():
                    wait_out(i - 2, 1)

            def do_slot(s):
                wait_in(i, s)
                src = ibufs[s]
                dst = obufs[s]
                rows_gw = rows * GW

                @plsc.parallel_loop(0, GW, unroll=8)
                def _(rm):
                    vals = plsc.load_gather(src, [rows_gw + rm])
                    dst[pl.ds(rm * D, D)] = vals

                start_out(i, s)

            @pl.when(slot == 0)
            def _():
                do_slot(0)

            @pl.when(slot == 1)
            def _():
                do_slot(1)

            return 0

        lax.fori_loop(0, n_i, grp, 0)

        def drain_at(i):
            @pl.when(lax.rem(i, 2) == 0)
            def _():
                wait_out(i, 0)

            @pl.when(lax.rem(i, 2) == 1)
            def _():
                wait_out(i, 1)

        drain_at(n_i - 2)
        drain_at(n_i - 1)

    main()

    @pl.when(wid == NW - 1)
    def _():
        pltpu.sync_copy(tail_hbm, tbuf)
        pltpu.sync_copy(tbuf, out_hbm.at[pl.ds(RFULL * D, TAIL * D)])


_reformat = functools.partial(
    pl.kernel,
    mesh=plsc.VectorSubcoreMesh(core_axis_name="c", subcore_axis_name="s"),
    out_type=jax.ShapeDtypeStruct((R * D,), jnp.float32),
    scratch_types=[
        pltpu.VMEM((L * GW,), jnp.float32),
        pltpu.VMEM((L * GW,), jnp.float32),
        pltpu.VMEM((GW * D,), jnp.float32),
        pltpu.VMEM((GW * D,), jnp.float32),
        pltpu.VMEM((TAIL * D,), jnp.float32),
        pltpu.SemaphoreType.DMA,
        pltpu.SemaphoreType.DMA,
        pltpu.SemaphoreType.DMA,
        pltpu.SemaphoreType.DMA,
    ],
    compiler_params=pltpu.CompilerParams(
        needs_layout_passes=False, use_tc_tiling_on_sc=True),
)(_reformat_body)


def _fm_body(idx_hbm, emb_hbm, lin_hbm, out_hbm,
             idx_v, rows_v, lin_v, acc_v, gsem, lsem):
    wid = lax.axis_index("s") * NC + lax.axis_index("c")
    base = wid * BPW

    def chunk(c, _):
        cbase = base + c * CH
        pltpu.sync_copy(idx_hbm.at[pl.ds(cbase * F, CF)], idx_v)

        @plsc.parallel_loop(0, NV, unroll=8)
        def offs(j):
            p = j * L + lax.iota(jnp.int32, L)
            f = lax.rem(p, F)
            idx_v[pl.ds(j * L, L)] = idx_v[pl.ds(j * L, L)] + f * V

        def fire(g, _):
            pltpu.async_copy(emb_hbm.at[idx_v.at[pl.ds(g * GS, GS)]],
                             rows_v.at[pl.ds(g * GS, GS)], gsem)
            pltpu.async_copy(lin_hbm.at[idx_v.at[pl.ds(g * GS, GS)]],
                             lin_v.at[pl.ds(g * GS, GS)], lsem)
            return 0

        lax.fori_loop(0, NG, fire, 0)

        zv = jnp.zeros((L,), jnp.float32)

        @plsc.parallel_loop(0, CH // L)
        def init(j):
            acc_v[pl.ds(j * L, L)] = zv

        def drain(g, _):
            pltpu.make_async_copy(emb_hbm.at[idx_v.at[pl.ds(g * GS, GS)]],
                                  rows_v.at[pl.ds(g * GS, GS)], gsem).wait()
            pltpu.make_async_copy(lin_hbm.at[idx_v.at[pl.ds(g * GS, GS)]],
                                  lin_v.at[pl.ds(g * GS, GS)], lsem).wait()
            return 0

        lax.fori_loop(0, NG, drain, 0)

        def lin_pass(j, _):
            p = j * L + lax.iota(jnp.int32, L)
            sid = lax.div(p, F)
            plsc.addupdate_scatter(acc_v, [sid], lin_v[pl.ds(j * L, L)])
            return 0

        lax.fori_loop(0, NV, lin_pass, 0)

        @plsc.parallel_loop(0, CH, unroll=2)
        def sample(b):
            s = jnp.zeros((L,), jnp.float32)
            q = jnp.zeros((L,), jnp.float32)
            for f in range(F):
                r = rows_v[b * F + f, :]
                s = s + r
                q = q + r * r
            t = (s * s - q) * 0.5
            sid = jnp.zeros((L,), jnp.int32) + b
            plsc.addupdate_scatter(acc_v, [sid], t)

        pltpu.sync_copy(acc_v, out_hbm.at[pl.ds(cbase, CH)])
        return 0

    lax.fori_loop(0, NCH, chunk, 0)


_fm = functools.partial(
    pl.kernel,
    mesh=plsc.VectorSubcoreMesh(core_axis_name="c", subcore_axis_name="s"),
    out_type=jax.ShapeDtypeStruct((B,), jnp.float32),
    scratch_types=[
        pltpu.VMEM((CF,), jnp.int32),
        pltpu.VMEM((CF, D), jnp.float32),
        pltpu.VMEM((CF,), jnp.float32),
        pltpu.VMEM((CH,), jnp.float32),
        pltpu.SemaphoreType.DMA,
        pltpu.SemaphoreType.DMA,
    ],
    compiler_params=pltpu.CompilerParams(
        needs_layout_passes=False, use_tc_tiling_on_sc=False),
)(_fm_body)


@jax.jit
def kernel(inputs, emb_table, lin_table, bias):
    idx_flat = inputs.astype(jnp.int32).reshape(B * F)
    lin_flat = lin_table.reshape(F * V)
    tail_rows = emb_table[RFULL:, :].reshape(TAIL * D)
    table_flat = _reformat(emb_table.T, tail_rows)
    table = table_flat.reshape(F * V, D)
    out = _fm(idx_flat, table, lin_flat)
    return out.reshape(B, 1) + bias
